# Initial kernel scaffold; baseline (speedup 1.0000x reference)
import jax
import jax.numpy as jnp
from jax.experimental import pallas as pl


def kernel(x_nchw, weight_oihw):
    raise NotImplementedError("write your pallas kernel here")


# speedup vs baseline: 12.4102x; 12.4102x over previous
import functools

import jax
import jax.numpy as jnp
from jax.experimental import pallas as pl
from jax.experimental.pallas import tpu as pltpu


def _conv_silu_kernel(p_ref, w_ref, o_ref, a_ref, *, k, cq, span):
    b, _, c1, _ = p_ref.shape
    for n in range(b):
        for kh in range(k):
            for kw in range(k):
                t = kh * k + kw
                q = (kh % 2) * 2 + (kw % 2)
                off = (kh // 2) * cq + (kw // 2)
                a_ref[t * c1:(t + 1) * c1, n * span:(n + 1) * span] = (
                    p_ref[n, q, :, off:off + span])

    acc = jnp.dot(w_ref[...], a_ref[...], preferred_element_type=jnp.float32)
    acc = acc * jax.nn.sigmoid(acc)
    for n in range(b):
        o_ref[n] = acc[:, n * span:(n + 1) * span]


@functools.partial(jax.jit, static_argnames=("k", "s", "p", "imgs_per_step"))
def _cov_act(x_nchw, weight_oihw, *, k, s, p, imgs_per_step):
    N, C1, H, W = x_nchw.shape
    C2 = weight_oihw.shape[0]
    Ho = (H + 2 * p - k) // s + 1
    Wo = (W + 2 * p - k) // s + 1
    KKC1 = k * k * C1

    CQ = (W + 2 * p) // 2
    RQ = Ho + 2
    span = Ho * CQ
    pad_bot = 2 * RQ - H - p
    x_pad = jnp.pad(x_nchw, ((0, 0), (0, 0), (p, pad_bot), (p, p)))
    phases = x_pad.reshape(N, C1, RQ, 2, CQ, 2)
    phases = phases.transpose(0, 3, 5, 1, 2, 4).reshape(N, 4, C1, RQ * CQ)
    phases = phases.astype(jnp.bfloat16)

    w2 = weight_oihw.transpose(0, 2, 3, 1).reshape(C2, KKC1)
    w2 = w2.astype(jnp.bfloat16)

    B = imgs_per_step
    cols = B * span
    body = functools.partial(_conv_silu_kernel, k=k, cq=CQ, span=span)
    cost = pl.CostEstimate(
        flops=2 * C2 * KKC1 * N * span,
        transcendentals=N * C2 * span,
        bytes_accessed=(N * 4 * C1 * RQ * CQ + C2 * KKC1) * 2 + N * C2 * span * 4,
    )
    out = pl.pallas_call(
        body,
        out_shape=jax.ShapeDtypeStruct((N, C2, span), jnp.float32),
        grid=(N // B,),
        in_specs=[
            pl.BlockSpec((B, 4, C1, RQ * CQ), lambda i: (i, 0, 0, 0)),
            pl.BlockSpec((C2, KKC1), lambda i: (0, 0)),
        ],
        out_specs=pl.BlockSpec((B, C2, span), lambda i: (i, 0, 0)),
        scratch_shapes=[pltpu.VMEM((KKC1, cols), jnp.bfloat16)],
        compiler_params=pltpu.CompilerParams(
            dimension_semantics=("parallel",),
            vmem_limit_bytes=64 << 20,
        ),
        cost_estimate=cost,
    )(phases, w2)

    return out.reshape(N, C2, Ho, CQ)[..., :Wo]


def kernel(x_nchw, weight_oihw):
    return _cov_act(x_nchw, weight_oihw, k=3, s=2, p=1, imgs_per_step=1)


# speedup vs baseline: 13.9350x; 1.1229x over previous
import functools

import jax
import jax.numpy as jnp
from jax.experimental import pallas as pl
from jax.experimental.pallas import tpu as pltpu


def _conv_silu_kernel(p_ref, w_ref, o_ref, a_ref, *, k, cq, span, ho, wo):
    b, _, c1, _ = p_ref.shape
    for n in range(b):
        for kh in range(k):
            for kw in range(k):
                t = kh * k + kw
                q = (kh % 2) * 2 + (kw % 2)
                off = (kh // 2) * cq + (kw // 2)
                a_ref[t * c1:(t + 1) * c1, n * span:(n + 1) * span] = (
                    p_ref[n, q, :, off:off + span])

    acc = jnp.dot(w_ref[...], a_ref[...], preferred_element_type=jnp.float32)
    acc = acc * jax.nn.sigmoid(acc)
    for n in range(b):
        for r in range(ho):
            o_ref[n, :, r * wo:(r + 1) * wo] = (
                acc[:, n * span + r * cq:n * span + r * cq + wo])


@functools.partial(jax.jit, static_argnames=("k", "s", "p", "imgs_per_step"))
def _cov_act(x_nchw, weight_oihw, *, k, s, p, imgs_per_step):
    N, C1, H, W = x_nchw.shape
    C2 = weight_oihw.shape[0]
    Ho = (H + 2 * p - k) // s + 1
    Wo = (W + 2 * p - k) // s + 1
    KKC1 = k * k * C1

    CQ = (W + 2 * p) // 2
    RQ = Ho + 2
    span = Ho * CQ
    pad_bot = 2 * RQ - H - p
    x_pad = jnp.pad(x_nchw.astype(jnp.bfloat16),
                    ((0, 0), (0, 0), (p, pad_bot), (p, p)))
    phases = x_pad.reshape(N, C1, RQ, 2, CQ, 2)
    phases = phases.transpose(0, 3, 5, 1, 2, 4).reshape(N, 4, C1, RQ * CQ)

    w2 = weight_oihw.transpose(0, 2, 3, 1).reshape(C2, KKC1)
    w2 = w2.astype(jnp.bfloat16)

    B = imgs_per_step
    cols = B * span
    body = functools.partial(_conv_silu_kernel, k=k, cq=CQ, span=span,
                             ho=Ho, wo=Wo)
    cost = pl.CostEstimate(
        flops=2 * C2 * KKC1 * N * span,
        transcendentals=N * C2 * span,
        bytes_accessed=(N * 4 * C1 * RQ * CQ + C2 * KKC1) * 2
        + N * C2 * Ho * Wo * 4,
    )
    out = pl.pallas_call(
        body,
        out_shape=jax.ShapeDtypeStruct((N, C2, Ho * Wo), jnp.float32),
        grid=(N // B,),
        in_specs=[
            pl.BlockSpec((B, 4, C1, RQ * CQ), lambda i: (i, 0, 0, 0)),
            pl.BlockSpec((C2, KKC1), lambda i: (0, 0)),
        ],
        out_specs=pl.BlockSpec((B, C2, Ho * Wo), lambda i: (i, 0, 0)),
        scratch_shapes=[pltpu.VMEM((KKC1, cols), jnp.bfloat16)],
        compiler_params=pltpu.CompilerParams(
            dimension_semantics=("parallel",),
            vmem_limit_bytes=64 << 20,
        ),
        cost_estimate=cost,
    )(phases, w2)

    return out.reshape(N, C2, Ho, Wo)


def kernel(x_nchw, weight_oihw):
    return _cov_act(x_nchw, weight_oihw, k=3, s=2, p=1, imgs_per_step=1)


# speedup vs baseline: 15.2289x; 1.0929x over previous
import functools

import jax
import jax.numpy as jnp
from jax.experimental import pallas as pl
from jax.experimental.pallas import tpu as pltpu


def _conv_silu_kernel(p_ref, w_ref, o_ref, a_ref, *, k, cq, span, ho, wo):
    b, _, c1, _ = p_ref.shape
    for n in range(b):
        for kh in range(k):
            for kw in range(k):
                t = kh * k + kw
                q = ((kh + 1) % 2) * 2 + (kw + 1) % 2
                off = (0 if kh == 0 else 1) * cq + (0 if kw == 0 else 1)
                a_ref[t * c1:(t + 1) * c1, n * span:(n + 1) * span] = (
                    p_ref[n, q, :, off:off + span])

    acc = jnp.dot(w_ref[...], a_ref[...], preferred_element_type=jnp.float32)
    acc = acc * jax.nn.sigmoid(acc)
    for n in range(b):
        for r in range(ho):
            o_ref[n, :, r * wo:(r + 1) * wo] = (
                acc[:, n * span + r * cq:n * span + r * cq + wo])


@functools.partial(jax.jit, static_argnames=("k", "s", "p", "imgs_per_step"))
def _cov_act(x_nchw, weight_oihw, *, k, s, p, imgs_per_step):
    N, C1, H, W = x_nchw.shape
    C2 = weight_oihw.shape[0]
    Ho = (H + 2 * p - k) // s + 1
    Wo = (W + 2 * p - k) // s + 1
    KKC1 = k * k * C1

    RI, CI = H // 2, W // 2
    RQ, CQ = RI + 2, CI + 2
    span = Ho * CQ
    u = x_nchw.astype(jnp.bfloat16).reshape(N, C1, RI, 2, CI, 2)
    u = u.transpose(0, 3, 5, 1, 2, 4)
    phases = jnp.pad(u, ((0, 0), (0, 0), (0, 0), (0, 0), (1, 1), (1, 1)))
    phases = phases.reshape(N, 4, C1, RQ * CQ)

    w2 = weight_oihw.transpose(0, 2, 3, 1).reshape(C2, KKC1)
    w2 = w2.astype(jnp.bfloat16)

    B = imgs_per_step
    cols = B * span
    body = functools.partial(_conv_silu_kernel, k=k, cq=CQ, span=span,
                             ho=Ho, wo=Wo)
    cost = pl.CostEstimate(
        flops=2 * C2 * KKC1 * N * span,
        transcendentals=N * C2 * span,
        bytes_accessed=(N * 4 * C1 * RQ * CQ + C2 * KKC1) * 2
        + N * C2 * Ho * Wo * 4,
    )
    out = pl.pallas_call(
        body,
        out_shape=jax.ShapeDtypeStruct((N, C2, Ho * Wo), jnp.float32),
        grid=(N // B,),
        in_specs=[
            pl.BlockSpec((B, 4, C1, RQ * CQ), lambda i: (i, 0, 0, 0)),
            pl.BlockSpec((C2, KKC1), lambda i: (0, 0)),
        ],
        out_specs=pl.BlockSpec((B, C2, Ho * Wo), lambda i: (i, 0, 0)),
        scratch_shapes=[pltpu.VMEM((KKC1, cols), jnp.bfloat16)],
        compiler_params=pltpu.CompilerParams(
            dimension_semantics=("parallel",),
            vmem_limit_bytes=64 << 20,
        ),
        cost_estimate=cost,
    )(phases, w2)

    return out.reshape(N, C2, Ho, Wo)


def kernel(x_nchw, weight_oihw):
    return _cov_act(x_nchw, weight_oihw, k=3, s=2, p=1, imgs_per_step=1)


# speedup vs baseline: 16.4128x; 1.0777x over previous
import functools

import jax
import jax.numpy as jnp
from jax.experimental import pallas as pl
from jax.experimental.pallas import tpu as pltpu


def _conv_silu_kernel(x_ref, w_ref, o_ref, eo_ref, p_ref, a_ref, *,
                      k, c1, ri, cq, span, ho, wo):
    b = x_ref.shape[0]
    lanes = x_ref.shape[3]
    ncols = lanes // 2
    rq = ri + 2

    l_io = jax.lax.broadcasted_iota(jnp.int32, (lanes, 256), 0)
    o_io = jax.lax.broadcasted_iota(jnp.int32, (lanes, 256), 1)
    q_o = o_io // cq
    s_o = o_io % cq - 1
    l_want = (q_o // 2) * ncols + 2 * s_o + (q_o % 2)
    valid = (o_io < 4 * cq) & (o_io % cq >= 1) & (s_o < ncols // 2)
    sel = jnp.where(valid & (l_io == l_want), 1.0, 0.0).astype(jnp.bfloat16)

    p_ref[:, :, 0:cq] = jnp.zeros((4, c1, cq), jnp.bfloat16)
    p_ref[:, :, (rq - 1) * cq:rq * cq] = jnp.zeros((4, c1, cq), jnp.bfloat16)

    for n in range(b):
        eo = jnp.dot(x_ref[n].reshape(ri * c1, lanes), sel,
                     preferred_element_type=jnp.float32)
        eo_ref[...] = eo.astype(jnp.bfloat16).reshape(ri, c1, 256)

        for r in range(ri):
            for q in range(4):
                p_ref[q, :, (1 + r) * cq:(2 + r) * cq] = (
                    eo_ref[r, :, q * cq:(q + 1) * cq])

        for kh in range(k):
            for kw in range(k):
                t = kh * k + kw
                q = ((kh + 1) % 2) * 2 + (kw + 1) % 2
                off = (0 if kh == 0 else 1) * cq + (0 if kw == 0 else 1)
                a_ref[t * c1:(t + 1) * c1, n * span:(n + 1) * span] = (
                    p_ref[q, :, off:off + span])

    acc = jnp.dot(w_ref[...], a_ref[...], preferred_element_type=jnp.float32)
    acc = acc * jax.nn.sigmoid(acc)
    for n in range(b):
        for r in range(ho):
            o_ref[n, :, r * wo:(r + 1) * wo] = (
                acc[:, n * span + r * cq:n * span + r * cq + wo])


@functools.partial(jax.jit, static_argnames=("k", "s", "p", "imgs_per_step"))
def _cov_act(x_nchw, weight_oihw, *, k, s, p, imgs_per_step):
    N, C1, H, W = x_nchw.shape
    C2 = weight_oihw.shape[0]
    Ho = (H + 2 * p - k) // s + 1
    Wo = (W + 2 * p - k) // s + 1
    KKC1 = k * k * C1
    RI, CI = H // 2, W // 2
    RQ, CQ = RI + 2, CI + 2
    span = Ho * CQ

    x_sr = x_nchw.astype(jnp.bfloat16).reshape(N, C1, RI, 2 * W)
    x_sr = x_sr.transpose(0, 2, 1, 3)

    w2 = weight_oihw.transpose(0, 2, 3, 1).reshape(C2, KKC1)
    w2 = w2.astype(jnp.bfloat16)

    B = imgs_per_step
    body = functools.partial(_conv_silu_kernel, k=k, c1=C1, ri=RI, cq=CQ,
                             span=span, ho=Ho, wo=Wo)
    cost = pl.CostEstimate(
        flops=2 * C2 * KKC1 * N * span + 2 * N * RI * C1 * 2 * W * 256,
        transcendentals=N * C2 * span,
        bytes_accessed=(N * C1 * H * W + C2 * KKC1) * 2 + N * C2 * Ho * Wo * 4,
    )
    out = pl.pallas_call(
        body,
        out_shape=jax.ShapeDtypeStruct((N, C2, Ho * Wo), jnp.float32),
        grid=(N // B,),
        in_specs=[
            pl.BlockSpec((B, RI, C1, 2 * W), lambda i: (i, 0, 0, 0)),
            pl.BlockSpec((C2, KKC1), lambda i: (0, 0)),
        ],
        out_specs=pl.BlockSpec((B, C2, Ho * Wo), lambda i: (i, 0, 0)),
        scratch_shapes=[
            pltpu.VMEM((RI, C1, 256), jnp.bfloat16),
            pltpu.VMEM((4, C1, RQ * CQ), jnp.bfloat16),
            pltpu.VMEM((KKC1, B * span), jnp.bfloat16),
        ],
        compiler_params=pltpu.CompilerParams(
            dimension_semantics=("parallel",),
            vmem_limit_bytes=64 << 20,
        ),
        cost_estimate=cost,
    )(x_sr, w2)

    return out.reshape(N, C2, Ho, Wo)


def kernel(x_nchw, weight_oihw):
    return _cov_act(x_nchw, weight_oihw, k=3, s=2, p=1, imgs_per_step=1)


# speedup vs baseline: 16.4254x; 1.0008x over previous
import functools

import jax
import jax.numpy as jnp
from jax.experimental import pallas as pl
from jax.experimental.pallas import tpu as pltpu


def _conv_silu_kernel(x_ref, w_ref, o_ref, eo_ref, p_ref, a_ref, *,
                      k, c1, ri, cq, span, ho, wo):
    b = x_ref.shape[0]
    lanes = x_ref.shape[3]
    ncols = lanes // 2
    rq = ri + 2

    l_io = jax.lax.broadcasted_iota(jnp.int32, (lanes, 256), 0)
    o_io = jax.lax.broadcasted_iota(jnp.int32, (lanes, 256), 1)
    q_o = o_io // cq
    s_o = o_io % cq - 1
    l_want = (q_o // 2) * ncols + 2 * s_o + (q_o % 2)
    valid = (o_io < 4 * cq) & (o_io % cq >= 1) & (s_o < ncols // 2)
    sel = jnp.where(valid & (l_io == l_want), 1.0, 0.0).astype(jnp.bfloat16)

    p_ref[:, :, 0:cq] = jnp.zeros((4, c1, cq), jnp.bfloat16)
    p_ref[:, :, (rq - 1) * cq:rq * cq] = jnp.zeros((4, c1, cq), jnp.bfloat16)

    for n in range(b):
        eo = jnp.dot(x_ref[n].reshape(ri * c1, lanes), sel,
                     preferred_element_type=jnp.float32)
        eo_ref[...] = eo.astype(jnp.bfloat16).reshape(ri, c1, 256)

        for r in range(ri):
            for q in range(4):
                p_ref[q, :, (1 + r) * cq:(2 + r) * cq] = (
                    eo_ref[r, :, q * cq:(q + 1) * cq])

        for kh in range(k):
            for kw in range(k):
                t = kh * k + kw
                q = ((kh + 1) % 2) * 2 + (kw + 1) % 2
                off = (0 if kh == 0 else 1) * cq + (0 if kw == 0 else 1)
                a_ref[t * c1:(t + 1) * c1, n * span:(n + 1) * span] = (
                    p_ref[q, :, off:off + span])

    acc = jnp.dot(w_ref[...], a_ref[...], preferred_element_type=jnp.float32)
    acc = acc * jax.nn.sigmoid(acc)
    for n in range(b):
        for r in range(ho):
            o_ref[n, :, r * wo:(r + 1) * wo] = (
                acc[:, n * span + r * cq:n * span + r * cq + wo])


@functools.partial(jax.jit, static_argnames=("k", "s", "p", "imgs_per_step"))
def _cov_act(x_nchw, weight_oihw, *, k, s, p, imgs_per_step):
    N, C1, H, W = x_nchw.shape
    C2 = weight_oihw.shape[0]
    Ho = (H + 2 * p - k) // s + 1
    Wo = (W + 2 * p - k) // s + 1
    KKC1 = k * k * C1
    RI, CI = H // 2, W // 2
    RQ, CQ = RI + 2, CI + 2
    span = Ho * CQ

    x_sr = x_nchw.reshape(N, C1, RI, 2 * W)
    x_sr = x_sr.transpose(0, 2, 1, 3).astype(jnp.bfloat16)

    w2 = weight_oihw.transpose(0, 2, 3, 1).reshape(C2, KKC1)
    w2 = w2.astype(jnp.bfloat16)

    B = imgs_per_step
    body = functools.partial(_conv_silu_kernel, k=k, c1=C1, ri=RI, cq=CQ,
                             span=span, ho=Ho, wo=Wo)
    cost = pl.CostEstimate(
        flops=2 * C2 * KKC1 * N * span + 2 * N * RI * C1 * 2 * W * 256,
        transcendentals=N * C2 * span,
        bytes_accessed=(N * C1 * H * W + C2 * KKC1) * 2 + N * C2 * Ho * Wo * 4,
    )
    out = pl.pallas_call(
        body,
        out_shape=jax.ShapeDtypeStruct((N, C2, Ho * Wo), jnp.float32),
        grid=(N // B,),
        in_specs=[
            pl.BlockSpec((B, RI, C1, 2 * W), lambda i: (i, 0, 0, 0)),
            pl.BlockSpec((C2, KKC1), lambda i: (0, 0)),
        ],
        out_specs=pl.BlockSpec((B, C2, Ho * Wo), lambda i: (i, 0, 0)),
        scratch_shapes=[
            pltpu.VMEM((RI, C1, 256), jnp.bfloat16),
            pltpu.VMEM((4, C1, RQ * CQ), jnp.bfloat16),
            pltpu.VMEM((KKC1, B * span), jnp.bfloat16),
        ],
        compiler_params=pltpu.CompilerParams(
            dimension_semantics=("parallel",),
            vmem_limit_bytes=64 << 20,
        ),
        cost_estimate=cost,
    )(x_sr, w2)

    return out.reshape(N, C2, Ho, Wo)


def kernel(x_nchw, weight_oihw):
    return _cov_act(x_nchw, weight_oihw, k=3, s=2, p=1, imgs_per_step=1)


# speedup vs baseline: 16.6457x; 1.0134x over previous
import functools

import jax
import jax.numpy as jnp
from jax.experimental import pallas as pl
from jax.experimental.pallas import tpu as pltpu


def _conv_silu_kernel(x_ref, w_ref, o_ref, eo_ref, p_ref, a_ref, *,
                      k, c1, ri, cq, span, ho, wo):
    b = x_ref.shape[0]
    lanes = x_ref.shape[3]
    ncols = lanes // 2
    rq = ri + 2

    l_io = jax.lax.broadcasted_iota(jnp.int32, (lanes, 256), 0)
    o_io = jax.lax.broadcasted_iota(jnp.int32, (lanes, 256), 1)
    q_o = o_io // cq
    s_o = o_io % cq - 1
    l_want = (q_o // 2) * ncols + 2 * s_o + (q_o % 2)
    valid = (o_io < 4 * cq) & (o_io % cq >= 1) & (s_o < ncols // 2)
    sel = jnp.where(valid & (l_io == l_want), 1.0, 0.0).astype(jnp.bfloat16)

    p_ref[:, :, 0:cq] = jnp.zeros((4, c1, cq), jnp.bfloat16)
    p_ref[:, :, (rq - 1) * cq:rq * cq] = jnp.zeros((4, c1, cq), jnp.bfloat16)

    for n in range(b):
        eo = jnp.dot(x_ref[n].reshape(ri * c1, lanes), sel,
                     preferred_element_type=jnp.float32)
        eo_ref[...] = eo.astype(jnp.bfloat16).reshape(ri, c1, 256)

        for r in range(ri):
            for q in range(4):
                p_ref[q, :, (1 + r) * cq:(2 + r) * cq] = (
                    eo_ref[r, :, q * cq:(q + 1) * cq])

        for kh in range(k):
            for kw in range(k):
                t = kh * k + kw
                q = ((kh + 1) % 2) * 2 + (kw + 1) % 2
                off = (0 if kh == 0 else 1) * cq + (0 if kw == 0 else 1)
                a_ref[t * c1:(t + 1) * c1, n * span:(n + 1) * span] = (
                    p_ref[q, :, off:off + span])

    acc = jnp.dot(w_ref[...], a_ref[...], preferred_element_type=jnp.float32)
    acc = acc * jax.nn.sigmoid(acc)
    for n in range(b):
        for r in range(ho):
            o_ref[n, :, r * wo:(r + 1) * wo] = (
                acc[:, n * span + r * cq:n * span + r * cq + wo])


@functools.partial(jax.jit, static_argnames=("k", "s", "p", "imgs_per_step"))
def _cov_act(x_nchw, weight_oihw, *, k, s, p, imgs_per_step):
    N, C1, H, W = x_nchw.shape
    C2 = weight_oihw.shape[0]
    Ho = (H + 2 * p - k) // s + 1
    Wo = (W + 2 * p - k) // s + 1
    KKC1 = k * k * C1
    RI, CI = H // 2, W // 2
    RQ, CQ = RI + 2, CI + 2
    span = Ho * CQ

    x_sr = x_nchw.reshape(N, C1, RI, 2 * W)
    x_sr = x_sr.transpose(0, 2, 1, 3).astype(jnp.bfloat16)

    w2 = weight_oihw.transpose(0, 2, 3, 1).reshape(C2, KKC1)
    w2 = w2.astype(jnp.bfloat16)

    B = imgs_per_step
    body = functools.partial(_conv_silu_kernel, k=k, c1=C1, ri=RI, cq=CQ,
                             span=span, ho=Ho, wo=Wo)
    cost = pl.CostEstimate(
        flops=2 * C2 * KKC1 * N * span + 2 * N * RI * C1 * 2 * W * 256,
        transcendentals=N * C2 * span,
        bytes_accessed=(N * C1 * H * W + C2 * KKC1) * 2 + N * C2 * Ho * Wo * 4,
    )
    out = pl.pallas_call(
        body,
        out_shape=jax.ShapeDtypeStruct((N, C2, Ho * Wo), jnp.float32),
        grid=(N // B,),
        in_specs=[
            pl.BlockSpec((B, RI, C1, 2 * W), lambda i: (i, 0, 0, 0)),
            pl.BlockSpec((C2, KKC1), lambda i: (0, 0)),
        ],
        out_specs=pl.BlockSpec((B, C2, Ho * Wo), lambda i: (i, 0, 0)),
        scratch_shapes=[
            pltpu.VMEM((RI, C1, 256), jnp.bfloat16),
            pltpu.VMEM((4, C1, RQ * CQ), jnp.bfloat16),
            pltpu.VMEM((KKC1, B * span), jnp.bfloat16),
        ],
        compiler_params=pltpu.CompilerParams(
            dimension_semantics=("parallel",),
            vmem_limit_bytes=64 << 20,
        ),
        cost_estimate=cost,
    )(x_sr, w2)

    return out.reshape(N, C2, Ho, Wo)


def kernel(x_nchw, weight_oihw):
    return _cov_act(x_nchw, weight_oihw, k=3, s=2, p=1, imgs_per_step=2)


# speedup vs baseline: 16.9139x; 1.0161x over previous
import functools

import jax
import jax.numpy as jnp
from jax.experimental import pallas as pl
from jax.experimental.pallas import tpu as pltpu


def _conv_silu_kernel(x_ref, w_ref, o_ref, eo_ref, p_ref, a_ref, *,
                      k, c1, ri, cq, span, ho, wo):
    b = x_ref.shape[0]
    lanes = x_ref.shape[3]
    ncols = lanes // 2
    rq = ri + 2

    l_io = jax.lax.broadcasted_iota(jnp.int32, (lanes, 256), 0)
    o_io = jax.lax.broadcasted_iota(jnp.int32, (lanes, 256), 1)
    q_o = o_io // cq
    s_o = o_io % cq - 1
    l_want = (q_o // 2) * ncols + 2 * s_o + (q_o % 2)
    valid = (o_io < 4 * cq) & (o_io % cq >= 1) & (s_o < ncols // 2)
    sel = jnp.where(valid & (l_io == l_want), 1.0, 0.0).astype(jnp.bfloat16)

    p_ref[:, :, 0:cq] = jnp.zeros((4, c1, cq), jnp.bfloat16)
    p_ref[:, :, (rq - 1) * cq:rq * cq] = jnp.zeros((4, c1, cq), jnp.bfloat16)

    for n in range(b):
        eo = jnp.dot(x_ref[n].astype(jnp.bfloat16).reshape(ri * c1, lanes),
                     sel, preferred_element_type=jnp.float32)
        eo_ref[...] = eo.astype(jnp.bfloat16).reshape(ri, c1, 256)

        for r in range(ri):
            for q in range(4):
                p_ref[q, :, (1 + r) * cq:(2 + r) * cq] = (
                    eo_ref[r, :, q * cq:(q + 1) * cq])

        for kh in range(k):
            for kw in range(k):
                t = kh * k + kw
                q = ((kh + 1) % 2) * 2 + (kw + 1) % 2
                off = (0 if kh == 0 else 1) * cq + (0 if kw == 0 else 1)
                a_ref[t * c1:(t + 1) * c1, n * span:(n + 1) * span] = (
                    p_ref[q, :, off:off + span])

    acc = jnp.dot(w_ref[...], a_ref[...], preferred_element_type=jnp.float32)
    acc = acc * jax.nn.sigmoid(acc)
    for n in range(b):
        for r in range(ho):
            o_ref[n, :, r * wo:(r + 1) * wo] = (
                acc[:, n * span + r * cq:n * span + r * cq + wo])


@functools.partial(jax.jit, static_argnames=("k", "s", "p", "imgs_per_step"))
def _cov_act(x_nchw, weight_oihw, *, k, s, p, imgs_per_step):
    N, C1, H, W = x_nchw.shape
    C2 = weight_oihw.shape[0]
    Ho = (H + 2 * p - k) // s + 1
    Wo = (W + 2 * p - k) // s + 1
    KKC1 = k * k * C1
    RI, CI = H // 2, W // 2
    RQ, CQ = RI + 2, CI + 2
    span = Ho * CQ

    x_sr = x_nchw.reshape(N, C1, RI, 2 * W)
    x_sr = x_sr.transpose(0, 2, 1, 3)

    w2 = weight_oihw.transpose(0, 2, 3, 1).reshape(C2, KKC1)
    w2 = w2.astype(jnp.bfloat16)

    B = imgs_per_step
    body = functools.partial(_conv_silu_kernel, k=k, c1=C1, ri=RI, cq=CQ,
                             span=span, ho=Ho, wo=Wo)
    cost = pl.CostEstimate(
        flops=2 * C2 * KKC1 * N * span + 2 * N * RI * C1 * 2 * W * 256,
        transcendentals=N * C2 * span,
        bytes_accessed=(N * C1 * H * W + C2 * KKC1) * 2 + N * C2 * Ho * Wo * 4,
    )
    out = pl.pallas_call(
        body,
        out_shape=jax.ShapeDtypeStruct((N, C2, Ho * Wo), jnp.float32),
        grid=(N // B,),
        in_specs=[
            pl.BlockSpec((B, RI, C1, 2 * W), lambda i: (i, 0, 0, 0)),
            pl.BlockSpec((C2, KKC1), lambda i: (0, 0)),
        ],
        out_specs=pl.BlockSpec((B, C2, Ho * Wo), lambda i: (i, 0, 0)),
        scratch_shapes=[
            pltpu.VMEM((RI, C1, 256), jnp.bfloat16),
            pltpu.VMEM((4, C1, RQ * CQ), jnp.bfloat16),
            pltpu.VMEM((KKC1, B * span), jnp.bfloat16),
        ],
        compiler_params=pltpu.CompilerParams(
            dimension_semantics=("parallel",),
            vmem_limit_bytes=64 << 20,
        ),
        cost_estimate=cost,
    )(x_sr, w2)

    return out.reshape(N, C2, Ho, Wo)


def kernel(x_nchw, weight_oihw):
    return _cov_act(x_nchw, weight_oihw, k=3, s=2, p=1, imgs_per_step=2)
